# baseline (speedup 1.0000x reference)
import functools

import jax
import jax.numpy as jnp
from jax import lax
from jax.experimental import pallas as pl
from jax.experimental.pallas import tpu as pltpu
from jax.experimental.pallas import tpu_sc as plsc

N_EDGES = 320000
N_NODES = 10000
D = 128
NC = 2
NS = 16
NW = NC * NS
B_PER_W = N_EDGES // NW
C = 16
CHUNKS = B_PER_W // C
NSLOT = 8
NBUF = 4
LANES = 16


def _edge_sym_body(tbl, src, dst, out, stbl,
                   si0, si1, si2, si3, si4, si5, si6, si7,
                   di0, di1, di2, di3, di4, di5, di6, di7,
                   ti0, ti1, ti2, ti3, tj0, tj1, tj2, tj3,
                   ob0, ob1, ob2, ob3,
                   is0, is1, is2, is3, is4, is5, is6, is7,
                   gs0, gs1, gs2, gs3, ss0, ss1, ss2, ss3):
    si = [si0, si1, si2, si3, si4, si5, si6, si7]
    di = [di0, di1, di2, di3, di4, di5, di6, di7]
    ti = [ti0, ti1, ti2, ti3]
    tj = [tj0, tj1, tj2, tj3]
    ob = [ob0, ob1, ob2, ob3]
    isem = [is0, is1, is2, is3, is4, is5, is6, is7]
    gsem = [gs0, gs1, gs2, gs3]
    ssem = [ss0, ss1, ss2, ss3]

    sid = lax.axis_index("s")
    wid = sid * NC + lax.axis_index("c")
    base = wid * B_PER_W

    row0 = sid * 624
    pltpu.sync_copy(tbl.at[pl.ds(row0, 624)], stbl.at[pl.ds(row0, 624)])

    @pl.when(sid == NS - 1)
    def _():
        pltpu.sync_copy(tbl.at[pl.ds(624 * NS, N_NODES - 624 * NS)],
                        stbl.at[pl.ds(624 * NS, N_NODES - 624 * NS)])

    plsc.subcore_barrier()

    def idx_copies(s, g):
        off = base + g * C
        c1 = pltpu.make_async_copy(src.at[pl.ds(off, C)], si[s], isem[s])
        c2 = pltpu.make_async_copy(dst.at[pl.ds(off, C)], di[s], isem[s])
        return c1, c2

    def issue_idx(s, g):
        c1, c2 = idx_copies(s, g)
        c1.start()
        c2.start()

    def wait_idx(s, g):
        c1, c2 = idx_copies(s, g)
        c1.wait()
        c2.wait()

    def gather_copies(v, s):
        c1 = pltpu.make_async_copy(stbl.at[si[s]], ti[v], gsem[v])
        c2 = pltpu.make_async_copy(stbl.at[di[s]], tj[v], gsem[v])
        return c1, c2

    def issue_gathers(v, s):
        c1, c2 = gather_copies(v, s)
        c1.start()
        c2.start()

    def wait_gathers(v, s):
        c1, c2 = gather_copies(v, s)
        c1.wait()
        c2.wait()

    def issue_store(v, g):
        off = base + g * C
        pltpu.make_async_copy(ob[v], out.at[pl.ds(off, C)], ssem[v]).start()

    def wait_store(v):
        pltpu.make_async_copy(ob[v], out.at[pl.ds(base, C)], ssem[v]).wait()

    for q in range(NSLOT):
        issue_idx(q, q)
    for q in range(NBUF):
        wait_idx(q, q)
        issue_gathers(q, q)

    def outer(g4, carry):
        for b in range(NSLOT):
            g = g4 * NSLOT + b
            v = b % NBUF

            @pl.when(g < CHUNKS)
            def _process():
                wait_gathers(v, b)

                @pl.when(g + NSLOT < CHUNKS)
                def _():
                    issue_idx(b, g + NSLOT)

                @pl.when(g >= NBUF)
                def _():
                    wait_store(v)

                @plsc.parallel_loop(0, C, unroll=4)
                def _rows(i):
                    for j in range(D // LANES):
                        a = ti[v][i, pl.ds(j * LANES, LANES)]
                        bb = tj[v][i, pl.ds(j * LANES, LANES)]
                        ob[v][i, pl.ds(j * LANES, LANES)] = a + bb
                        ob[v][i, pl.ds(D + j * LANES, LANES)] = a - bb

                issue_store(v, g)

                @pl.when(g + NBUF < CHUNKS)
                def _():
                    wait_idx((b + NBUF) % NSLOT, g + NBUF)
                    issue_gathers(v, (b + NBUF) % NSLOT)
        return carry

    lax.fori_loop(0, (CHUNKS + NSLOT - 1) // NSLOT, outer, 0)
    for v in range(NBUF):
        wait_store(v)


_edge_sym = functools.partial(
    pl.kernel,
    mesh=plsc.VectorSubcoreMesh(core_axis_name="c", subcore_axis_name="s"),
    out_type=jax.ShapeDtypeStruct((N_EDGES, 2 * D), jnp.float32),
    scratch_types=[
        pltpu.VMEM_SHARED((N_NODES, D), jnp.float32),
        *([pltpu.VMEM((C,), jnp.int32)] * 16),
        *([pltpu.VMEM((C, D), jnp.float32)] * 8),
        *([pltpu.VMEM((C, 2 * D), jnp.float32)] * 4),
        *([pltpu.SemaphoreType.DMA] * 16),
    ],
)(_edge_sym_body)


def kernel(node_attrs, edge_index):
    ei = edge_index.astype(jnp.int32)
    return _edge_sym(node_attrs, ei[0], ei[1])


# speedup vs baseline: 1.4500x; 1.4500x over previous
import functools

import jax
import jax.numpy as jnp
from jax import lax
from jax.experimental import pallas as pl
from jax.experimental.pallas import tpu as pltpu
from jax.experimental.pallas import tpu_sc as plsc

N_EDGES = 320000
N_NODES = 10000
D = 128
NC = 2
NS = 16
NW = NC * NS
B_PER_W = N_EDGES // NW
C = 16
CHUNKS = B_PER_W // C
NBUF = 3
LANES = 16


def _edge_sym_body(tbl, src, dst, out, stbl, sidx, didx,
                   ti0, ti1, ti2, tj0, tj1, tj2, ob0, ob1, ob2,
                   gs0, gs1, gs2, ss0, ss1, ss2):
    ti = [ti0, ti1, ti2]
    tj = [tj0, tj1, tj2]
    ob = [ob0, ob1, ob2]
    gsem = [gs0, gs1, gs2]
    ssem = [ss0, ss1, ss2]

    sid = lax.axis_index("s")
    wid = sid * NC + lax.axis_index("c")
    base = wid * B_PER_W
    pltpu.sync_copy(src.at[pl.ds(base, B_PER_W)], sidx)
    pltpu.sync_copy(dst.at[pl.ds(base, B_PER_W)], didx)

    row0 = sid * 624
    pltpu.sync_copy(tbl.at[pl.ds(row0, 624)], stbl.at[pl.ds(row0, 624)])

    @pl.when(sid == NS - 1)
    def _():
        pltpu.sync_copy(tbl.at[pl.ds(624 * NS, N_NODES - 624 * NS)],
                        stbl.at[pl.ds(624 * NS, N_NODES - 624 * NS)])

    plsc.subcore_barrier()

    def gather_copies(b, g):
        lo = g * C
        c1 = pltpu.make_async_copy(stbl.at[sidx.at[pl.ds(lo, C)]], ti[b], gsem[b])
        c2 = pltpu.make_async_copy(stbl.at[didx.at[pl.ds(lo, C)]], tj[b], gsem[b])
        return c1, c2

    def issue_gathers(b, g):
        c1, c2 = gather_copies(b, g)
        c1.start()
        c2.start()

    def wait_gathers(b, g):
        c1, c2 = gather_copies(b, g)
        c1.wait()
        c2.wait()

    def issue_store(b, g):
        off = base + g * C
        pltpu.make_async_copy(ob[b], out.at[pl.ds(off, C)], ssem[b]).start()

    def wait_store(b):
        pltpu.make_async_copy(ob[b], out.at[pl.ds(base, C)], ssem[b]).wait()

    for b0 in range(NBUF):
        issue_gathers(b0, b0)

    def outer(g2, carry):
        for b in range(NBUF):
            g = g2 * NBUF + b

            @pl.when(g < CHUNKS)
            def _process():
                wait_gathers(b, g)

                @pl.when(g >= NBUF)
                def _():
                    wait_store(b)

                @plsc.parallel_loop(0, C, unroll=4)
                def _rows(i):
                    for j in range(D // LANES):
                        a = ti[b][i, pl.ds(j * LANES, LANES)]
                        bb = tj[b][i, pl.ds(j * LANES, LANES)]
                        ob[b][i, pl.ds(j * LANES, LANES)] = a + bb
                        ob[b][i, pl.ds(D + j * LANES, LANES)] = a - bb
                issue_store(b, g)

                @pl.when(g + NBUF < CHUNKS)
                def _():
                    issue_gathers(b, g + NBUF)
        return carry

    lax.fori_loop(0, (CHUNKS + NBUF - 1) // NBUF, outer, 0)
    for b in range(NBUF):
        wait_store(b)


_edge_sym = functools.partial(
    pl.kernel,
    mesh=plsc.VectorSubcoreMesh(core_axis_name="c", subcore_axis_name="s"),
    out_type=jax.ShapeDtypeStruct((N_EDGES, 2 * D), jnp.float32),
    scratch_types=[
        pltpu.VMEM_SHARED((N_NODES, D), jnp.float32),
        pltpu.VMEM((B_PER_W,), jnp.int32),
        pltpu.VMEM((B_PER_W,), jnp.int32),
        *([pltpu.VMEM((C, D), jnp.float32)] * 6),
        *([pltpu.VMEM((C, 2 * D), jnp.float32)] * 3),
        *([pltpu.SemaphoreType.DMA] * 6),
    ],
)(_edge_sym_body)


def kernel(node_attrs, edge_index):
    ei = edge_index.astype(jnp.int32)
    return _edge_sym(node_attrs, ei[0], ei[1])
